# baseline (speedup 1.0000x reference)
import functools
import numpy as np
import jax
import jax.numpy as jnp
from jax import lax
from jax.experimental import pallas as pl
from jax.experimental.pallas import tpu as pltpu
from jax.experimental.pallas import tpu_sc as plsc

B, H, W = 2, 512, 512
MAX_KPTS = 1024
NUM_SCALES = 3
DIFF_ITERS = 3
KAPPA = 0.05
THRESHOLD = 0.001
AKAZE_NMS = 5
ORI_PATCH = 15
ORI_SIGMA = 2.5
NUM_PAIRS = 256
SINK_ITERS = 20
EPSILON = 1.0
UNUSED = 1.0
NMS_RADIUS = 3
SCORE_THRESH = 0.0
MAX_RADIUS = 16
BORDER = MAX_RADIUS

NEG = -1e30
NPAD = 1152


def _shift1e(x, d, axis):
    if axis == 0:
        if d == 1:
            return jnp.concatenate([x[1:, :], x[-1:, :]], axis=0)
        return jnp.concatenate([x[:1, :], x[:-1, :]], axis=0)
    if d == 1:
        return jnp.concatenate([x[:, 1:], x[:, -1:]], axis=1)
    return jnp.concatenate([x[:, :1], x[:, :-1]], axis=1)


def _shift(x, dy, dx):
    if dy != 0:
        x = _shift1e(x, dy, 0)
    if dx != 0:
        x = _shift1e(x, dx, 1)
    return x


def _maxpool(x, radius):
    up = x
    dn = x
    m = x
    for _ in range(radius):
        up = _shift1e(up, 1, 0)
        dn = _shift1e(dn, -1, 0)
        m = jnp.maximum(m, jnp.maximum(up, dn))
    up = m
    dn = m
    out = m
    for _ in range(radius):
        up = _shift1e(up, 1, 1)
        dn = _shift1e(dn, -1, 1)
        out = jnp.maximum(out, jnp.maximum(up, dn))
    return out


def _diffusion_step(L):
    Le = _shift(L, 0, 1)
    Lw = _shift(L, 0, -1)
    Ls = _shift(L, 1, 0)
    Ln = _shift(L, -1, 0)
    Lx = 0.5 * (Le - Lw)
    Ly = 0.5 * (Ls - Ln)
    g = jnp.exp(-((Lx * Lx + Ly * Ly) / (KAPPA * KAPPA)))
    fe = Le - L
    fw = L - Lw
    fs = Ls - L
    fn = L - Ln
    ge = 0.5 * (g + _shift(g, 0, 1))
    gw = 0.5 * (g + _shift(g, 0, -1))
    gs = 0.5 * (g + _shift(g, 1, 0))
    gn = 0.5 * (g + _shift(g, -1, 0))
    return L + 0.25 * (ge * fe - gw * fw + gs * fs - gn * fn)


def _hessian_response(L):
    Le = _shift(L, 0, 1)
    Lw = _shift(L, 0, -1)
    Ls = _shift(L, 1, 0)
    Ln = _shift(L, -1, 0)
    Lxx = Le - 2.0 * L + Lw
    Lyy = Ls - 2.0 * L + Ln
    Lxy = 0.25 * (_shift(L, 1, 1) - _shift(L, 1, -1)
                  - _shift(L, -1, 1) + _shift(L, -1, -1))
    return Lxx * Lyy - Lxy * Lxy


def _shiftz(x, d, axis):
    if d == 0:
        return x
    n = abs(d)
    if axis == 0:
        z = jnp.zeros((n, x.shape[1]), x.dtype)
        if d > 0:
            return jnp.concatenate([x[n:, :], z], axis=0)
        return jnp.concatenate([z, x[:-n, :]], axis=0)
    z = jnp.zeros((x.shape[0], n), x.dtype)
    if d > 0:
        return jnp.concatenate([x[:, n:], z], axis=1)
    return jnp.concatenate([z, x[:, :-n]], axis=1)


_rr = ORI_PATCH // 2
_ax = np.arange(-_rr, _rr + 1, dtype=np.float64)
_g1 = np.exp(-(_ax * _ax) / (2.0 * ORI_SIGMA * ORI_SIGMA)).astype(np.float32)
_W1D = (_g1 / _g1.sum()).tolist()


def _gauss_smooth(x):
    for axis in (1, 0):
        acc = _W1D[_rr] * x
        for t in range(1, _rr + 1):
            acc = acc + _W1D[_rr + t] * (_shiftz(x, t, axis)
                                         + _shiftz(x, -t, axis))
        x = acc
    return x


def _akaze_body(x_ref, ms_ref, c_ref, s_ref):
    L = x_ref[0]
    scores = jnp.zeros_like(L)
    for _s in range(NUM_SCALES):
        L = lax.fori_loop(0, DIFF_ITERS, lambda i, Lc: _diffusion_step(Lc), L)
        r = _hessian_response(L)
        keep = (r >= _maxpool(r, AKAZE_NMS // 2)) & (r > THRESHOLD)
        scores = jnp.maximum(scores, jnp.where(keep, r, 0.0))

    Lx = 0.5 * (_shift(L, 0, 1) - _shift(L, 0, -1))
    Ly = 0.5 * (_shift(L, 1, 0) - _shift(L, -1, 0))
    sx = _gauss_smooth(Lx)
    sy = _gauss_smooth(Ly)
    rn = jnp.sqrt(sx * sx + sy * sy)
    safe = rn > 0.0
    rs = jnp.maximum(rn, 1e-30)
    c_ref[0] = jnp.where(safe, sx / rs, 1.0)
    s_ref[0] = jnp.where(safe, sy / rs, 0.0)

    nms = scores >= _maxpool(scores, NMS_RADIUS)
    yy = lax.broadcasted_iota(jnp.int32, (H, W), 0)
    xx = lax.broadcasted_iota(jnp.int32, (H, W), 1)
    bm = ((yy >= BORDER) & (yy < H - BORDER)
          & (xx >= BORDER) & (xx < W - BORDER))
    valid = nms & (scores > SCORE_THRESH) & bm
    ms_ref[0] = jnp.where(valid, scores, -jnp.inf)


def _akaze_all(x):
    n = x.shape[0]
    out = jax.ShapeDtypeStruct((n, H, W), jnp.float32)
    return pl.pallas_call(
        _akaze_body,
        grid=(n,),
        in_specs=[pl.BlockSpec((1, H, W), lambda i: (i, 0, 0))],
        out_specs=[pl.BlockSpec((1, H, W), lambda i: (i, 0, 0))] * 3,
        out_shape=[out, out, out],
    )(x)


def _sink_body(d1_ref, d2_ref, out_ref):
    d1 = d1_ref[0]
    d2 = d2_ref[0]
    G = lax.dot_general(d1, d2, (((1,), (1,)), ((), ())),
                        preferred_element_type=jnp.float32)
    sq1 = jnp.sum(d1 * d1, axis=1, keepdims=True)
    sq2 = jnp.sum(d2 * d2, axis=1, keepdims=True)
    d2m = sq1 + jnp.transpose(sq2) - 2.0 * G
    sc = -jnp.sqrt(jnp.clip(d2m, 0.0, None) + 1e-12)

    scp = jnp.concatenate(
        [jnp.concatenate([sc, jnp.zeros((MAX_KPTS, NPAD - MAX_KPTS),
                                        jnp.float32)], axis=1),
         jnp.zeros((NPAD - MAX_KPTS, NPAD), jnp.float32)], axis=0)
    ri = lax.broadcasted_iota(jnp.int32, (NPAD, NPAD), 0)
    ci = lax.broadcasted_iota(jnp.int32, (NPAD, NPAD), 1)
    main = (ri < MAX_KPTS) & (ci < MAX_KPTS)
    bins = (((ri == MAX_KPTS) & (ci <= MAX_KPTS))
            | ((ci == MAX_KPTS) & (ri <= MAX_KPTS)))
    Z = jnp.where(main, scp, jnp.where(bins, UNUSED, NEG)) / EPSILON

    norm = -np.log(2.0 * MAX_KPTS)
    li = lax.broadcasted_iota(jnp.int32, (NPAD, 1), 0)
    log_mu = jnp.where(li < MAX_KPTS, norm,
                       jnp.where(li == MAX_KPTS,
                                 np.log(float(MAX_KPTS)) + norm, NEG))
    log_nu = jnp.transpose(log_mu)

    def body(_i, uv):
        u, v = uv
        t = Z + v
        m = jnp.max(t, axis=1, keepdims=True)
        u = log_mu - (m + jnp.log(jnp.sum(jnp.exp(t - m), axis=1,
                                          keepdims=True)))
        t = Z + u
        m = jnp.max(t, axis=0, keepdims=True)
        v = log_nu - (m + jnp.log(jnp.sum(jnp.exp(t - m), axis=0,
                                          keepdims=True)))
        return (u, v)

    u, v = lax.fori_loop(0, SINK_ITERS, body,
                         (jnp.zeros((NPAD, 1), jnp.float32),
                          jnp.zeros((1, NPAD), jnp.float32)))
    out_ref[0] = jnp.exp(Z + u + v - norm)


def _sink_all(d1, d2):
    return pl.pallas_call(
        _sink_body,
        grid=(B,),
        in_specs=[pl.BlockSpec((1, MAX_KPTS, NUM_PAIRS), lambda i: (i, 0, 0)),
                  pl.BlockSpec((1, MAX_KPTS, NUM_PAIRS), lambda i: (i, 0, 0))],
        out_specs=pl.BlockSpec((1, NPAD, NPAD), lambda i: (i, 0, 0)),
        out_shape=jax.ShapeDtypeStruct((B, NPAD, NPAD), jnp.float32),
    )(d1, d2)


_NC, _NS = 2, 16
_NW = _NC * _NS


@functools.lru_cache(maxsize=None)
def _sc_gather_kernel(n_total, chunk, n_active):
    n_w = n_total // n_active
    n_chunks = n_w // chunk
    sub = chunk // 128
    mesh = plsc.VectorSubcoreMesh(core_axis_name="c", subcore_axis_name="s")

    def body(tbl_hbm, ridx_hbm, out_hbm, ridx_v, out_v, sem):
        wid = lax.axis_index("s") * _NC + lax.axis_index("c")
        base = wid * n_w

        def chunk_body(ci, carry):
            off = pl.multiple_of(base + ci * chunk, chunk)
            roff = pl.multiple_of(off // 128, sub)
            pltpu.sync_copy(ridx_hbm.at[pl.ds(roff, sub), :], ridx_v)
            cps = [pltpu.async_copy(tbl_hbm.at[ridx_v.at[j]],
                                    out_v.at[pl.ds(j * 128, 128)], sem)
                   for j in range(sub)]
            for cp in cps:
                cp.wait()
            pltpu.sync_copy(out_v, out_hbm.at[pl.ds(off, chunk)])
            return carry

        @pl.when(wid < n_active)
        def _():
            lax.fori_loop(0, n_chunks, chunk_body, 0)

    return pl.kernel(
        body,
        out_type=jax.ShapeDtypeStruct((n_total,), jnp.float32),
        mesh=mesh,
        scratch_types=[
            pltpu.VMEM((sub, 128), jnp.int32),
            pltpu.VMEM((chunk,), jnp.float32),
            pltpu.SemaphoreType.DMA,
        ],
    )


def _sc_gather(table_flat, idx):
    n = int(idx.shape[0])
    n_active = min(_NW, n // 1024)
    chunk = min(2048, n // n_active)
    rows = idx.reshape(n // 128, 128)
    return _sc_gather_kernel(n, chunk, n_active)(table_flat, rows)


def _descriptors(imgs, kpts, cmap, smap, offsets, thresholds):
    nm = imgs.shape[0]
    hw = H * W
    mofs = (jnp.arange(nm, dtype=jnp.int32) * hw)[:, None]
    ky = kpts[..., 0]
    kx = kpts[..., 1]
    yi = jnp.clip(jnp.round(ky), 0, H - 1).astype(jnp.int32)
    xi = jnp.clip(jnp.round(kx), 0, W - 1).astype(jnp.int32)

    cs_tbl = jnp.concatenate([cmap.reshape(-1), smap.reshape(-1)])
    cidx = (yi * W + xi + mofs).reshape(-1)
    tidx = jnp.concatenate([cidx, cidx + nm * hw])
    cs = _sc_gather(cs_tbl, tidx)
    c = cs[:nm * MAX_KPTS].reshape(nm, MAX_KPTS)[..., None]
    s = cs[nm * MAX_KPTS:].reshape(nm, MAX_KPTS)[..., None]

    ox1, oy1, ox2, oy2 = (offsets[:, 0], offsets[:, 1],
                          offsets[:, 2], offsets[:, 3])
    rx1 = c * ox1 - s * oy1
    ry1 = s * ox1 + c * oy1
    rx2 = c * ox2 - s * oy2
    ry2 = s * ox2 + c * oy2
    y1 = jnp.clip(jnp.round(ky[..., None] + ry1), 0, H - 1).astype(jnp.int32)
    x1 = jnp.clip(jnp.round(kx[..., None] + rx1), 0, W - 1).astype(jnp.int32)
    y2 = jnp.clip(jnp.round(ky[..., None] + ry2), 0, H - 1).astype(jnp.int32)
    x2 = jnp.clip(jnp.round(kx[..., None] + rx2), 0, W - 1).astype(jnp.int32)
    f1 = (y1 * W + x1 + mofs[:, :, None]).reshape(-1)
    f2 = (y2 * W + x2 + mofs[:, :, None]).reshape(-1)
    ns = f1.shape[0]
    vals = _sc_gather(imgs.reshape(-1), jnp.concatenate([f1, f2]))
    v1 = vals[:ns].reshape(nm, MAX_KPTS, NUM_PAIRS)
    v2 = vals[ns:].reshape(nm, MAX_KPTS, NUM_PAIRS)
    desc = v1 - v2 - thresholds
    return desc / (jnp.linalg.norm(desc, axis=-1, keepdims=True) + 1e-8)


def kernel(image1, image2, pair_offsets, pair_thresholds):
    X = jnp.concatenate([image1[:, 0], image2[:, 0]], axis=0)
    ms, cmap, smap = _akaze_all(X)

    vals, idx = lax.top_k(ms.reshape(2 * B, -1), MAX_KPTS)
    ys = idx // W
    xs = idx % W
    ok = jnp.isfinite(vals)
    kpts = jnp.where(ok[..., None], jnp.stack([ys, xs], -1), -1)
    kpts = kpts.astype(jnp.float32)

    desc = _descriptors(X, kpts, cmap, smap, pair_offsets, pair_thresholds)
    probs_pad = _sink_all(desc[:B], desc[B:])
    probs = probs_pad[:, :MAX_KPTS + 1, :MAX_KPTS + 1]
    return kpts[:B], kpts[B:], probs
